# baseline (speedup 1.0000x reference)
import jax
import jax.numpy as jnp
from jax.experimental import pallas as pl

_B = 64


def _body(x_ref, o_ref):
    o_ref[...] = x_ref[:, 50:150, :] - x_ref[:, 0:1, :]


def kernel(x, W):
    del W
    n, dim, d = x.shape
    return pl.pallas_call(
        _body,
        grid=(n // _B,),
        in_specs=[pl.BlockSpec((_B, dim, d), lambda i: (i, 0, 0))],
        out_specs=pl.BlockSpec((_B, 100, d), lambda i: (i, 0, 0)),
        out_shape=jax.ShapeDtypeStruct((n, 100, d), x.dtype),
    )(x)


# speedup vs baseline: 9.6133x; 9.6133x over previous
import jax
import jax.numpy as jnp
from jax.experimental import pallas as pl

_B = 512


def _body(off_ref, x_ref, o_ref):
    o_ref[...] = x_ref[...] - off_ref[...]


def kernel(x, W):
    del W
    n, dim, d = x.shape
    xt = jnp.transpose(x, (1, 2, 0))

    grid = (n // _B, 2)
    out_t = pl.pallas_call(
        _body,
        grid=grid,
        in_specs=[
            pl.BlockSpec((1, d, _B), lambda i, j: (0, 0, i)),
            pl.BlockSpec((50, d, _B), lambda i, j: (j + 1, 0, i)),
        ],
        out_specs=pl.BlockSpec((50, d, _B), lambda i, j: (j, 0, i)),
        out_shape=jax.ShapeDtypeStruct((100, d, n), x.dtype),
    )(xt, xt)
    return jnp.transpose(out_t, (2, 0, 1))


# speedup vs baseline: 9.7009x; 1.0091x over previous
import jax
import jax.numpy as jnp
from jax.experimental import pallas as pl

_B = 1024


def _body(off_ref, x_ref, o_ref):
    o_ref[...] = x_ref[...] - off_ref[...]


def kernel(x, W):
    del W
    n, dim, d = x.shape
    xt = jnp.transpose(x, (1, 2, 0))

    grid = (n // _B, 2)
    out_t = pl.pallas_call(
        _body,
        grid=grid,
        in_specs=[
            pl.BlockSpec((1, d, _B), lambda i, j: (0, 0, i)),
            pl.BlockSpec((50, d, _B), lambda i, j: (j + 1, 0, i)),
        ],
        out_specs=pl.BlockSpec((50, d, _B), lambda i, j: (j, 0, i)),
        out_shape=jax.ShapeDtypeStruct((100, d, n), x.dtype),
    )(xt, xt)
    return jnp.transpose(out_t, (2, 0, 1))
